# Initial kernel scaffold; baseline (speedup 1.0000x reference)
import jax
import jax.numpy as jnp
from jax.experimental import pallas as pl


def kernel(in_W, out_W, out_B, in_num_bases, in_num_zero_vecs, in_num_w1, in_num_bias1, attr_W, num_attr_groups_mask, init_cat_idxs, init_numvals, init_nummasks, attr_idxs, domain_idxs, domain_masks):
    raise NotImplementedError("write your pallas kernel here")


# speedup vs baseline: 6.9277x; 6.9277x over previous
import jax
import jax.numpy as jnp
from jax import lax
from jax.experimental import pallas as pl
from jax.experimental.pallas import tpu as pltpu
from jax.experimental.pallas import tpu_sc as plsc

B = 4096
E = 64
N_CAT = 26
N_NUM = 13
N_GROUPS = 4
N_ATTRS = 39
D = 100

NC = 2
NS = 16
NW = NC * NS

CHUNK = 128
NBUF = 4

_SC_PARAMS = pltpu.CompilerParams(use_tc_tiling_on_sc=False)


def _emit_chunk_pipeline(table, idx_v, out_hbm, wbase, n_chunks,
                         rows_b, gsem, wsem,
                         bias_src=None, bias_out=None, bias_b=None, bsem=None):
    def group_ops(j0):
        gds, bds, wds = [], [], []
        for b in range(NBUF):
            sl = idx_v.at[pl.ds((j0 + b) * CHUNK, CHUNK)]
            gds.append(pltpu.async_copy(table.at[sl], rows_b.at[b], gsem.at[b]))
            if bias_src is not None:
                bds.append(pltpu.async_copy(bias_src.at[sl], bias_b.at[b],
                                            bsem.at[b]))
        for b in range(NBUF):
            gds[b].wait()
            base = wbase + (j0 + b) * CHUNK
            wds.append(pltpu.async_copy(rows_b.at[b],
                                        out_hbm.at[pl.ds(base, CHUNK)],
                                        wsem.at[b]))
            if bias_src is not None:
                bds[b].wait()
                wds.append(pltpu.async_copy(bias_b.at[b],
                                            bias_out.at[pl.ds(base, CHUNK)],
                                            bsem.at[b]))
        for w in wds:
            w.wait()

    full = n_chunks // NBUF

    def group(jo, _):
        group_ops(jo * NBUF)
        return _

    lax.fori_loop(0, full, group, None)

    for t in range(n_chunks % NBUF):
        j = full * NBUF + t
        sl = idx_v.at[pl.ds(j * CHUNK, CHUNK)]
        pltpu.async_copy(table.at[sl], rows_b.at[0], gsem.at[0]).wait()
        base = wbase + j * CHUNK
        pltpu.async_copy(rows_b.at[0], out_hbm.at[pl.ds(base, CHUNK)],
                         wsem.at[0]).wait()
        if bias_src is not None:
            pltpu.async_copy(bias_src.at[sl], bias_b.at[0], bsem.at[0]).wait()
            pltpu.async_copy(bias_b.at[0], bias_out.at[pl.ds(base, CHUNK)],
                             bsem.at[0]).wait()


def _sc_gather_body(in_W, out_W, out_Bf, cat_idx, dom_idx,
                    cat_rows, dom_rows, dom_bias,
                    idx_cat_v, idx_dom_v, rows_b, bias_b, gsem, wsem, bsem):
    wid = lax.axis_index("s") * NC + lax.axis_index("c")

    n_cat_w = (B * N_CAT) // NW
    n_dom_w = (B * D) // NW

    pltpu.sync_copy(cat_idx.at[pl.ds(wid * n_cat_w, n_cat_w)], idx_cat_v)
    pltpu.sync_copy(dom_idx.at[pl.ds(wid * n_dom_w, n_dom_w)], idx_dom_v)

    _emit_chunk_pipeline(in_W, idx_cat_v, cat_rows, wid * n_cat_w,
                         n_cat_w // CHUNK, rows_b, gsem, wsem)
    _emit_chunk_pipeline(out_W, idx_dom_v, dom_rows, wid * n_dom_w,
                         n_dom_w // CHUNK, rows_b, gsem, wsem,
                         bias_src=out_Bf, bias_out=dom_bias,
                         bias_b=bias_b, bsem=bsem)


@jax.jit
def _sc_gather(in_W, out_W, out_B, cat_idx_flat, dom_idx_flat):
    mesh = plsc.VectorSubcoreMesh(core_axis_name="c", subcore_axis_name="s")
    return pl.kernel(
        _sc_gather_body,
        out_type=(
            jax.ShapeDtypeStruct((B * N_CAT, E), jnp.float32),
            jax.ShapeDtypeStruct((B * D, E), jnp.float32),
            jax.ShapeDtypeStruct((B * D,), jnp.float32),
        ),
        mesh=mesh,
        compiler_params=_SC_PARAMS,
        scratch_types=[
            pltpu.VMEM(((B * N_CAT) // NW,), jnp.int32),
            pltpu.VMEM(((B * D) // NW,), jnp.int32),
            pltpu.VMEM((NBUF, CHUNK, E), jnp.float32),
            pltpu.VMEM((NBUF, CHUNK), jnp.float32),
            pltpu.SemaphoreType.DMA((NBUF,)),
            pltpu.SemaphoreType.DMA((NBUF,)),
            pltpu.SemaphoreType.DMA((NBUF,)),
        ],
    )(in_W, out_W, out_B.reshape(-1), cat_idx_flat, dom_idx_flat)


BLK = 128


def _tc_body(cat_ref, numvals_ref, nummasks_ref, bases_ref, zero_ref, w1_ref,
             bias1_ref, attrW_ref, gm_ref, attr_idx_ref, dom_ref, dom_bias_ref,
             dom_masks_ref, out_ref):
    numvals = numvals_ref[...]
    nummasks = nummasks_ref[...]
    gm = gm_ref[...]
    bases = bases_ref[...]

    nvs = []
    for g in range(N_GROUPS):
        mv = numvals * gm[g][None, :]
        h = jnp.dot(mv, bases, preferred_element_type=jnp.float32)
        h = jnp.maximum(h + zero_ref[g][None, :], 0.0)
        h = jnp.dot(h, w1_ref[g], preferred_element_type=jnp.float32)
        h = h + bias1_ref[g][None, :]
        full = jnp.sum(gm[g])
        got = jnp.sum(nummasks * gm[g][None, :], axis=1)
        h = h * (full == got).astype(jnp.float32)[:, None]
        nvs.append(h[:, None, :])
    nv = jnp.concatenate(nvs, axis=1)

    iv = jnp.concatenate([cat_ref[...], nv], axis=1)
    norm = jnp.maximum(jnp.sqrt(jnp.sum(iv * iv, axis=2)), 1e-12)

    aidx = attr_idx_ref[...]
    onehot = (lax.broadcasted_iota(jnp.int32, (BLK, N_ATTRS), 1)
              == aidx).astype(jnp.float32)
    alog = jnp.dot(onehot, attrW_ref[...],
                   preferred_element_type=jnp.float32)
    m = jnp.max(alog, axis=1, keepdims=True)
    e = jnp.exp(alog - m)
    w = e / jnp.sum(e, axis=1, keepdims=True)
    w = w / norm

    combined = jnp.sum(w[:, :, None] * iv, axis=1)

    dot = jnp.sum(dom_ref[...] * combined[:, None, :], axis=2)
    out_ref[...] = dot + dom_bias_ref[...] + dom_masks_ref[...]


@jax.jit
def _tc_dense(cat_rows, init_numvals, init_nummasks, in_num_bases,
              in_num_zero_vecs, in_num_w1, in_num_bias1, attr_W,
              num_attr_groups_mask, attr_idxs, dom_rows, dom_bias,
              domain_masks):
    grid = (B // BLK,)

    def bb(*shape):
        return pl.BlockSpec(shape, lambda i: (i,) + (0,) * (len(shape) - 1))

    def full(*shape):
        return pl.BlockSpec(shape, lambda i: (0,) * len(shape))

    return pl.pallas_call(
        _tc_body,
        grid=grid,
        in_specs=[
            bb(BLK, N_CAT, E),
            bb(BLK, N_NUM),
            bb(BLK, N_NUM),
            full(N_NUM, E),
            full(N_GROUPS, E),
            full(N_GROUPS, E, E),
            full(N_GROUPS, E),
            full(N_ATTRS, N_CAT + N_GROUPS),
            full(N_GROUPS, N_NUM),
            bb(BLK, 1),
            bb(BLK, D, E),
            bb(BLK, D),
            bb(BLK, D),
        ],
        out_specs=bb(BLK, D),
        out_shape=jax.ShapeDtypeStruct((B, D), jnp.float32),
    )(cat_rows, init_numvals, init_nummasks, in_num_bases, in_num_zero_vecs,
      in_num_w1, in_num_bias1, attr_W, num_attr_groups_mask, attr_idxs,
      dom_rows, dom_bias, domain_masks)


def kernel(in_W, out_W, out_B, in_num_bases, in_num_zero_vecs, in_num_w1,
           in_num_bias1, attr_W, num_attr_groups_mask, init_cat_idxs,
           init_numvals, init_nummasks, attr_idxs, domain_idxs, domain_masks):
    cat_rows, dom_rows, dom_bias = _sc_gather(
        in_W, out_W, out_B,
        init_cat_idxs.reshape(-1), domain_idxs.reshape(-1))
    return _tc_dense(
        cat_rows.reshape(B, N_CAT, E), init_numvals, init_nummasks,
        in_num_bases, in_num_zero_vecs, in_num_w1, in_num_bias1, attr_W,
        num_attr_groups_mask, attr_idxs, dom_rows.reshape(B, D, E),
        dom_bias.reshape(B, D), domain_masks)
